# baseline (speedup 1.0000x reference)
import jax
import jax.numpy as jnp
from jax.experimental import pallas as pl

CHUNK = 1024
NSTREAMS = 2
STEP = CHUNK * NSTREAMS


def _mask_t(w, x):
    logits_t = jax.lax.dot_general(
        w,
        x,
        dimension_numbers=(((1,), (1,)), ((), ())),
        preferred_element_type=jnp.float32,
        precision=jax.lax.Precision.DEFAULT,
    )
    return (logits_t > 0.0).astype(jnp.int32)


def _gate_block(w_ref, xa_ref, xb_ref, o_ref):
    w = w_ref[...]
    o_ref[:, 0:CHUNK] = _mask_t(w, xa_ref[...])
    o_ref[:, CHUNK:STEP] = _mask_t(w, xb_ref[...])


@jax.jit
def kernel(cls_hidden_states, gate_w):
    tokens, hidden = cls_hidden_states.shape
    num_experts = gate_w.shape[0]

    grid = (tokens // STEP,)
    mask_t = pl.pallas_call(
        _gate_block,
        grid=grid,
        in_specs=[
            pl.BlockSpec((num_experts, hidden), lambda i: (0, 0)),
            pl.BlockSpec((CHUNK, hidden), lambda i: (2 * i, 0)),
            pl.BlockSpec((CHUNK, hidden), lambda i: (2 * i + 1, 0)),
        ],
        out_specs=pl.BlockSpec((num_experts, STEP), lambda i: (0, i)),
        out_shape=jax.ShapeDtypeStruct((num_experts, tokens), jnp.int32),
    )(gate_w, cls_hidden_states, cls_hidden_states)
    return mask_t.T


# speedup vs baseline: 1.0487x; 1.0487x over previous
import jax
import jax.numpy as jnp
from jax.experimental import pallas as pl

TOKEN_BLOCK = 1024


def _gate_block(w_ref, x_ref, o_ref):
    logits_t = jax.lax.dot_general(
        w_ref[...],
        x_ref[...],
        dimension_numbers=(((1,), (1,)), ((), ())),
        preferred_element_type=jnp.float32,
        precision=jax.lax.Precision.DEFAULT,
    )
    o_ref[...] = (logits_t > 0.0).astype(jnp.int32)


@jax.jit
def kernel(cls_hidden_states, gate_w):
    tokens, hidden = cls_hidden_states.shape
    num_experts = gate_w.shape[0]

    grid = (tokens // TOKEN_BLOCK,)
    mask_t = pl.pallas_call(
        _gate_block,
        grid=grid,
        in_specs=[
            pl.BlockSpec((num_experts, hidden), lambda i: (0, 0)),
            pl.BlockSpec((TOKEN_BLOCK, hidden), lambda i: (i, 0)),
        ],
        out_specs=pl.BlockSpec((num_experts, TOKEN_BLOCK), lambda i: (0, i)),
        out_shape=jax.ShapeDtypeStruct((num_experts, tokens), jnp.int32),
    )(gate_w, cls_hidden_states)
    return mask_t.T
